# baseline (speedup 1.0000x reference)
import functools

import jax
import jax.numpy as jnp
from jax import lax
from jax.experimental import pallas as pl
from jax.experimental.pallas import tpu as pltpu
from jax.experimental.pallas import tpu_sc as plsc

E = 8
H = 1024
I = 2048
T = 32

BI = 1024
NJ = I // BI
NR = E * NJ
BIS = 512
NS = I // BIS

_NC = 2
_NSUB = 16
_L = 16


def _silu(x):
    return x * jax.nn.sigmoid(x)


@functools.partial(
    pl.kernel,
    out_type=jax.ShapeDtypeStruct((T, _L), jnp.float32),
    mesh=plsc.VectorSubcoreMesh(core_axis_name="c", subcore_axis_name="s"),
    scratch_types=[
        pltpu.VMEM((H,), jnp.float32),
        pltpu.VMEM((E, H), jnp.float32),
        pltpu.VMEM((_L,), jnp.float32),
    ],
)
def _router_sc(x_hbm, rw_hbm, out_hbm, xv, rwv, colv):
    t = lax.axis_index("s") * _NC + lax.axis_index("c")
    pltpu.sync_copy(x_hbm.at[t], xv)
    pltpu.sync_copy(rw_hbm, rwv)

    lanes = lax.iota(jnp.int32, _L)

    _gdn = lax.GatherDimensionNumbers(
        offset_dims=(), collapsed_slice_dims=(0,), start_index_map=(0,))

    def _perm(v, shift):
        idx = ((lanes + shift) % _L).reshape(_L, 1)
        return lax.gather(v, idx, _gdn, slice_sizes=(1,),
                          mode=lax.GatherScatterMode.PROMISE_IN_BOUNDS)

    def _lane_sum(v):
        for sh in (8, 4, 2, 1):
            v = v + _perm(v, sh)
        return v

    def _lane_max(v):
        for sh in (8, 4, 2, 1):
            v = jnp.maximum(v, _perm(v, sh))
        return v

    def _lane_min(v):
        for sh in (8, 4, 2, 1):
            v = jnp.minimum(v, _perm(v, sh))
        return v

    def body(i, accs):
        xc = xv[pl.ds(i * _L, _L)]
        return tuple(accs[e] + xc * rwv[e, pl.ds(i * _L, _L)]
                     for e in range(E))

    accs = lax.fori_loop(
        0, H // _L, body,
        tuple(jnp.zeros((_L,), jnp.float32) for _ in range(E)))

    lv = jnp.full((_L,), -jnp.inf, jnp.float32)
    for e in range(E):
        lv = jnp.where(lanes == e, _lane_sum(accs[e]), lv)
    m = _lane_max(lv)
    idx = _lane_min(jnp.where(lv == m, lanes, _L))
    score = 1.0 / (1.0 + jnp.exp(-m))
    colv[...] = jnp.where(lanes == idx, score, 0.0)
    pltpu.sync_copy(colv, out_hbm.at[t])


def _shared_body(x_ref, shg_ref, shu_ref, shd_ref, out_ref):
    k = pl.program_id(0)

    @pl.when(k == 0)
    def _init():
        out_ref[...] = jnp.zeros_like(out_ref)

    x = x_ref[...]
    g = lax.dot_general(x, shg_ref[...], (((1,), (1,)), ((), ())),
                        preferred_element_type=jnp.float32)
    u = lax.dot_general(x, shu_ref[...], (((1,), (1,)), ((), ())),
                        preferred_element_type=jnp.float32)
    a = _silu(g) * u
    out_ref[...] += lax.dot_general(a, shd_ref[...], (((1,), (1,)), ((), ())),
                                    preferred_element_type=jnp.float32)


def _experts_body(x_ref, sp_ref, shout_ref, gate_ref, up_ref, down_ref,
                  out_ref, scores_ref, sc_scratch):
    k = pl.program_id(0)

    @pl.when(k == 0)
    def _init():
        scT = sp_ref[...].T
        sc_scratch[...] = scT
        scores_ref[...] = scT[:E]
        out_ref[...] = shout_ref[...]

    e = k // NJ
    srow = sc_scratch[pl.ds(e, 1), :]
    xs = x_ref[...] * srow.T
    g = jnp.dot(xs, gate_ref[0], preferred_element_type=jnp.float32)
    u = jnp.dot(xs, up_ref[0], preferred_element_type=jnp.float32)
    a = u * _silu(g)
    out_ref[...] += jnp.dot(a, down_ref[0], preferred_element_type=jnp.float32)


def kernel(hidden_states, router_w, gate_up_proj, down_proj,
           sh_gate, sh_up, sh_down):
    x = hidden_states.reshape(-1, H)

    scores_pad = _router_sc(x, router_w)

    sh_out = pl.pallas_call(
        _shared_body,
        grid=(NS,),
        in_specs=[
            pl.BlockSpec((T, H), lambda k: (0, 0)),
            pl.BlockSpec((BIS, H), lambda k: (k, 0)),
            pl.BlockSpec((BIS, H), lambda k: (k, 0)),
            pl.BlockSpec((H, BIS), lambda k: (0, k)),
        ],
        out_specs=pl.BlockSpec((T, H), lambda k: (0, 0)),
        out_shape=jax.ShapeDtypeStruct((T, H), jnp.float32),
        compiler_params=pltpu.CompilerParams(
            dimension_semantics=("arbitrary",),
        ),
    )(x, sh_gate, sh_up, sh_down)

    out, scores = pl.pallas_call(
        _experts_body,
        grid=(NR,),
        in_specs=[
            pl.BlockSpec((T, H), lambda k: (0, 0)),
            pl.BlockSpec((T, _L), lambda k: (0, 0)),
            pl.BlockSpec((T, H), lambda k: (0, 0)),
            pl.BlockSpec((1, H, BI), lambda k: (k // NJ, 0, k % NJ)),
            pl.BlockSpec((1, H, BI), lambda k: (k // NJ, 0, NJ + k % NJ)),
            pl.BlockSpec((1, BI, H), lambda k: (k // NJ, k % NJ, 0)),
        ],
        out_specs=[
            pl.BlockSpec((T, H), lambda k: (0, 0)),
            pl.BlockSpec((E, T), lambda k: (0, 0)),
        ],
        out_shape=[
            jax.ShapeDtypeStruct((T, H), jnp.float32),
            jax.ShapeDtypeStruct((E, T), jnp.float32),
        ],
        scratch_shapes=[pltpu.VMEM((_L, T), jnp.float32)],
        compiler_params=pltpu.CompilerParams(
            dimension_semantics=("arbitrary",),
        ),
    )(x, scores_pad, sh_out, gate_up_proj, gate_up_proj, down_proj)

    return (out, scores)
